# Initial kernel scaffold; baseline (speedup 1.0000x reference)
import jax
import jax.numpy as jnp
from jax.experimental import pallas as pl


def kernel(positions, params):
    raise NotImplementedError("write your pallas kernel here")


# speedup vs baseline: 1.6867x; 1.6867x over previous
import functools
import numpy as np
import jax
import jax.numpy as jnp
from jax.experimental import pallas as pl
from jax.experimental.pallas import tpu as pltpu

_B, _N, _OUT, _K = 8, 4096, 256, 25
_M1, _M2, _M3, _M4 = 820, 205, 52, 13


def _fps_stage(x, y, z, m_out):
    b, n = x.shape
    lane = jax.lax.broadcasted_iota(jnp.int32, (b, n), 1)
    col1 = jax.lax.broadcasted_iota(jnp.int32, (b, m_out), 1)

    x0 = x[:, 0:1]
    y0 = y[:, 0:1]
    z0 = z[:, 0:1]
    dx = x - x0
    dy = y - y0
    dz = z - z0
    d2 = dx * dx + dy * dy + dz * dz

    ox = jnp.where(col1 == 0, x0, jnp.zeros((b, m_out), jnp.float32))
    oy = jnp.where(col1 == 0, y0, jnp.zeros((b, m_out), jnp.float32))
    oz = jnp.where(col1 == 0, z0, jnp.zeros((b, m_out), jnp.float32))

    def body(i, state):
        d2, ox, oy, oz = state
        mx = jnp.max(d2, axis=1, keepdims=True)
        far = jnp.min(jnp.where(d2 == mx, lane, n), axis=1, keepdims=True)
        onehot = lane == far
        px = jnp.sum(jnp.where(onehot, x, 0.0), axis=1, keepdims=True)
        py = jnp.sum(jnp.where(onehot, y, 0.0), axis=1, keepdims=True)
        pz = jnp.sum(jnp.where(onehot, z, 0.0), axis=1, keepdims=True)
        ddx = x - px
        ddy = y - py
        ddz = z - pz
        nd2 = ddx * ddx + ddy * ddy + ddz * ddz
        d2 = jnp.minimum(d2, nd2)
        colmask = col1 == i
        ox = jnp.where(colmask, px, ox)
        oy = jnp.where(colmask, py, oy)
        oz = jnp.where(colmask, pz, oz)
        return d2, ox, oy, oz

    _, ox, oy, oz = jax.lax.fori_loop(1, m_out, body, (d2, ox, oy, oz))
    return ox, oy, oz


def _fps_kernel(pt_ref, s1_ref, s2_ref, s3_ref, s4_ref):
    x = pt_ref[0]
    y = pt_ref[1]
    z = pt_ref[2]
    x1, y1, z1 = _fps_stage(x, y, z, _M1)
    s1_ref[0] = x1
    s1_ref[1] = y1
    s1_ref[2] = z1
    x2, y2, z2 = _fps_stage(x1, y1, z1, _M2)
    s2_ref[0] = x2
    s2_ref[1] = y2
    s2_ref[2] = z2
    x3, y3, z3 = _fps_stage(x2, y2, z2, _M3)
    s3_ref[0] = x3
    s3_ref[1] = y3
    s3_ref[2] = z3
    x4, y4, z4 = _fps_stage(x3, y3, z3, _M4)
    s4_ref[0] = x4
    s4_ref[1] = y4
    s4_ref[2] = z4


@jax.jit
def _fps_call(pos_t):
    f32 = jnp.float32
    return pl.pallas_call(
        _fps_kernel,
        out_shape=(
            jax.ShapeDtypeStruct((3, _B, _M1), f32),
            jax.ShapeDtypeStruct((3, _B, _M2), f32),
            jax.ShapeDtypeStruct((3, _B, _M3), f32),
            jax.ShapeDtypeStruct((3, _B, _M4), f32),
        ),
    )(pos_t)


def _mlp_eval(layers, x):
    inv = 1.0 / np.sqrt(1.0 + 1e-5)
    for L in layers:
        x = jnp.maximum(x @ L['W'] + L['b'], 0.0)
        x = L['g'] * (x * inv) + L['be']
    return x


def _knn25(pos, cent, r):
    d2 = (jnp.sum(cent ** 2, -1)[:, :, None]
          + jnp.sum(pos ** 2, -1)[:, None, :]
          - 2.0 * jnp.einsum('bmd,bnd->bmn', cent, pos))
    negd2, idx = jax.lax.top_k(-d2, _K)
    mask = (-negd2) <= (r * r)
    return idx, mask


def _sa_level(feat, pos, cent, r, layers):
    nidx, mask = _knn25(pos, cent, r)
    gather = jax.vmap(lambda a, i: a[i])
    npos = gather(pos, nidx)
    rel = npos - cent[:, :, None, :]
    if feat is None:
        h = rel
    else:
        nfeat = gather(feat, nidx)
        h = jnp.concatenate([nfeat, rel], axis=-1)
    h = _mlp_eval(layers, h)
    h = jnp.where(mask[..., None], h, -jnp.inf)
    return jnp.max(h, axis=2)


def kernel(positions, params):
    pos_t = jnp.transpose(positions, (2, 0, 1))
    s1, s2, s3, s4 = _fps_call(pos_t)
    p1 = jnp.transpose(s1, (1, 2, 0))
    p2 = jnp.transpose(s2, (1, 2, 0))
    p3 = jnp.transpose(s3, (1, 2, 0))
    p4 = jnp.transpose(s4, (1, 2, 0))

    f1 = _sa_level(None, positions, p1, 3.0, params['sa1'])
    f2 = _sa_level(f1, p1, p2, 4.0, params['sa2'])
    f3 = _sa_level(f2, p2, p3, 5.0, params['sa3'])
    f4 = _sa_level(f3, p3, p4, 7.0, params['sa4'])

    g = _mlp_eval(params['last'], jnp.concatenate([f4, p4], axis=-1))
    g = jnp.max(g, axis=1)
    return g @ params['linW'] + params['linb']


# speedup vs baseline: 7.6854x; 4.5564x over previous
import functools
import numpy as np
import jax
import jax.numpy as jnp
from jax import lax
from jax.experimental import pallas as pl
from jax.experimental.pallas import tpu as pltpu

_B, _N, _OUT, _K = 8, 4096, 256, 25
_KP = 32
_M1, _M2, _M3, _M4 = 820, 205, 52, 13
_BNINV = float(1.0 / np.sqrt(1.0 + 1e-5))
_INF = float(np.inf)


def _fps_stage(x, y, z, m_out):
    b, n = x.shape
    lane = lax.broadcasted_iota(jnp.int32, (b, n), 1)
    col1 = lax.broadcasted_iota(jnp.int32, (b, m_out), 1)

    x0, y0, z0 = x[:, 0:1], y[:, 0:1], z[:, 0:1]
    dx, dy, dz = x - x0, y - y0, z - z0
    d2 = dx * dx + dy * dy + dz * dz

    zero = jnp.zeros((b, m_out), jnp.float32)
    ox = jnp.where(col1 == 0, x0, zero)
    oy = jnp.where(col1 == 0, y0, zero)
    oz = jnp.where(col1 == 0, z0, zero)

    def body(i, state):
        d2, ox, oy, oz = state
        mx = jnp.max(d2, axis=1, keepdims=True)
        far = jnp.min(jnp.where(d2 == mx, lane, n), axis=1, keepdims=True)
        onehot = lane == far
        px = jnp.sum(jnp.where(onehot, x, 0.0), axis=1, keepdims=True)
        py = jnp.sum(jnp.where(onehot, y, 0.0), axis=1, keepdims=True)
        pz = jnp.sum(jnp.where(onehot, z, 0.0), axis=1, keepdims=True)
        ddx, ddy, ddz = x - px, y - py, z - pz
        nd2 = ddx * ddx + ddy * ddy + ddz * ddz
        d2 = jnp.minimum(d2, nd2)
        colmask = col1 == i
        ox = jnp.where(colmask, px, ox)
        oy = jnp.where(colmask, py, oy)
        oz = jnp.where(colmask, pz, oz)
        return d2, ox, oy, oz

    _, ox, oy, oz = lax.fori_loop(1, m_out, body, (d2, ox, oy, oz))
    return ox, oy, oz


def _fps_kernel(pt_ref, s1_ref, s2_ref, s3_ref, s4_ref):
    x, y, z = pt_ref[0], pt_ref[1], pt_ref[2]
    for m, ref in ((_M1, s1_ref), (_M2, s2_ref), (_M3, s3_ref), (_M4, s4_ref)):
        x, y, z = _fps_stage(x, y, z, m)
        ref[0], ref[1], ref[2] = x, y, z


def _fps_call(pos_t):
    f32 = jnp.float32
    return pl.pallas_call(
        _fps_kernel,
        out_shape=(
            jax.ShapeDtypeStruct((3, _B, _M1), f32),
            jax.ShapeDtypeStruct((3, _B, _M2), f32),
            jax.ShapeDtypeStruct((3, _B, _M3), f32),
            jax.ShapeDtypeStruct((3, _B, _M4), f32),
        ),
    )(pos_t)


def _prep_kernel(x_ref, c_ref, wf_ref, wr_ref, b_ref, g_ref, c1_ref, *, has_feat):
    if has_feat:
        feat = x_ref[0][:, :-3]
        posr = x_ref[0][:, -3:]
        g = jnp.dot(feat, wf_ref[...], preferred_element_type=jnp.float32)
        g = g + jnp.dot(posr, wr_ref[...], preferred_element_type=jnp.float32)
    else:
        posr = x_ref[0]
        g = jnp.dot(posr, wr_ref[...], preferred_element_type=jnp.float32)
    g_ref[0] = g + b_ref[...]
    c1_ref[0] = jnp.dot(c_ref[0], wr_ref[...], preferred_element_type=jnp.float32)


def _prep_call(xin, cent, wf, wr, b1, has_feat):
    bb, n, cin = xin.shape
    m = cent.shape[1]
    c1 = wr.shape[1]
    kfn = functools.partial(_prep_kernel, has_feat=has_feat)
    return pl.pallas_call(
        kfn,
        grid=(_B,),
        in_specs=[
            pl.BlockSpec((1, n, cin), lambda b: (b, 0, 0)),
            pl.BlockSpec((1, m, 3), lambda b: (b, 0, 0)),
            pl.BlockSpec(wf.shape, lambda b: (0, 0)),
            pl.BlockSpec(wr.shape, lambda b: (0, 0)),
            pl.BlockSpec(b1.shape, lambda b: (0, 0)),
        ],
        out_specs=(
            pl.BlockSpec((1, n, c1), lambda b: (b, 0, 0)),
            pl.BlockSpec((1, m, c1), lambda b: (b, 0, 0)),
        ),
        out_shape=(
            jax.ShapeDtypeStruct((bb, n, c1), jnp.float32),
            jax.ShapeDtypeStruct((bb, m, c1), jnp.float32),
        ),
    )(xin, cent, wf, wr, b1)


def _topk_kernel(c_ref, pt_ref, idx_ref, nd2_ref, *, n):
    b = pl.program_id(0)
    cb = c_ref[0]
    ptb = pt_ref[0]
    mb = cb.shape[0]
    cp = jnp.dot(cb, ptb, preferred_element_type=jnp.float32)
    csq = jnp.sum(cb * cb, axis=1, keepdims=True)
    psq = jnp.sum(ptb * ptb, axis=0, keepdims=True)
    dd = csq + psq - 2.0 * cp

    lane = lax.broadcasted_iota(jnp.int32, (mb, n), 1)
    colk = lax.broadcasted_iota(jnp.int32, (mb, _KP), 1)
    pool_i = jnp.zeros((mb, _KP), jnp.int32)
    pool_d = jnp.full((mb, _KP), _INF, jnp.float32)

    def body(r, state):
        dd, pool_i, pool_d = state
        mn = jnp.min(dd, axis=1, keepdims=True)
        fi = jnp.min(jnp.where(dd == mn, lane, jnp.int32(2 ** 30)),
                     axis=1, keepdims=True)
        dd = jnp.where(lane == fi, _INF, dd)
        colm = colk == r
        pool_d = jnp.where(colm, mn, pool_d)
        pool_i = jnp.where(colm, fi, pool_i)
        return dd, pool_i, pool_d

    _, pool_i, pool_d = lax.fori_loop(0, _K, body, (dd, pool_i, pool_d))
    idx_ref[0] = pool_i + b * n
    nd2_ref[0] = pool_d


def _topk_call(cent, pos_t, mblk):
    bb, m, _ = cent.shape
    n = pos_t.shape[2]
    nmb = (m + mblk - 1) // mblk
    kfn = functools.partial(_topk_kernel, n=n)
    return pl.pallas_call(
        kfn,
        grid=(_B, nmb),
        in_specs=[
            pl.BlockSpec((1, mblk, 3), lambda b, mi: (b, mi, 0)),
            pl.BlockSpec((1, 3, n), lambda b, mi: (b, 0, 0)),
        ],
        out_specs=(
            pl.BlockSpec((1, mblk, _KP), lambda b, mi: (b, mi, 0)),
            pl.BlockSpec((1, mblk, _KP), lambda b, mi: (b, mi, 0)),
        ),
        out_shape=(
            jax.ShapeDtypeStruct((bb, m, _KP), jnp.int32),
            jax.ShapeDtypeStruct((bb, m, _KP), jnp.float32),
        ),
    )(cent, pos_t)


def _mlp_kernel(gn_ref, c1_ref, nd2_ref, w2_ref, b2_ref, w3_ref, b3_ref,
                p1_ref, p2_ref, p3_ref, out_ref, *, r2):
    mb = c1_ref.shape[1]
    c1w = c1_ref.shape[2]
    gn = gn_ref[0]
    pre = gn - c1_ref[0][:, None, :]
    h = jnp.maximum(pre, 0.0).reshape(mb * _KP, c1w)
    g1, be1 = p1_ref[0:1], p1_ref[1:2]
    h = (h * _BNINV) * g1 + be1
    h = jnp.maximum(jnp.dot(h, w2_ref[...], preferred_element_type=jnp.float32)
                    + b2_ref[...], 0.0)
    h = (h * _BNINV) * p2_ref[0:1] + p2_ref[1:2]
    h = jnp.maximum(jnp.dot(h, w3_ref[...], preferred_element_type=jnp.float32)
                    + b3_ref[...], 0.0)
    h = (h * _BNINV) * p3_ref[0:1] + p3_ref[1:2]
    cout = h.shape[1]
    h = h.reshape(mb, _KP, cout)
    maskadd = jnp.where(nd2_ref[0] <= r2, 0.0, -_INF)
    h = h + maskadd
    out_ref[0] = jnp.max(h, axis=1)


def _mlp_call(gn, c1m, nd2, w2, b2, w3, b3, p1, p2, p3, r, mblk):
    bb, m, _, c1w = gn.shape
    cout = w3.shape[1]
    nmb = (m + mblk - 1) // mblk
    kfn = functools.partial(_mlp_kernel, r2=float(r * r))
    return pl.pallas_call(
        kfn,
        grid=(_B, nmb),
        in_specs=[
            pl.BlockSpec((1, mblk, _KP, c1w), lambda b, mi: (b, mi, 0, 0)),
            pl.BlockSpec((1, mblk, c1w), lambda b, mi: (b, mi, 0)),
            pl.BlockSpec((1, mblk, _KP, 1), lambda b, mi: (b, mi, 0, 0)),
            pl.BlockSpec(w2.shape, lambda b, mi: (0, 0)),
            pl.BlockSpec(b2.shape, lambda b, mi: (0, 0)),
            pl.BlockSpec(w3.shape, lambda b, mi: (0, 0)),
            pl.BlockSpec(b3.shape, lambda b, mi: (0, 0)),
            pl.BlockSpec(p1.shape, lambda b, mi: (0, 0)),
            pl.BlockSpec(p2.shape, lambda b, mi: (0, 0)),
            pl.BlockSpec(p3.shape, lambda b, mi: (0, 0)),
        ],
        out_specs=pl.BlockSpec((1, mblk, cout), lambda b, mi: (b, mi, 0)),
        out_shape=jax.ShapeDtypeStruct((bb, m, cout), jnp.float32),
    )(gn, c1m, nd2.reshape(bb, m, _KP, 1), w2, b2, w3, b3, p1, p2, p3)


def _head_kernel(f_ref, p_ref, w1_ref, b1_ref, w2_ref, b2_ref, w3_ref, b3_ref,
                 g1_ref, g2_ref, g3_ref, lw_ref, lb_ref, out_ref):
    x = jnp.concatenate([f_ref[...], p_ref[...]], axis=1)
    for w, b, p in ((w1_ref, b1_ref, g1_ref), (w2_ref, b2_ref, g2_ref),
                    (w3_ref, b3_ref, g3_ref)):
        x = jnp.maximum(jnp.dot(x, w[...], preferred_element_type=jnp.float32)
                        + b[...], 0.0)
        x = (x * _BNINV) * p[0:1] + p[1:2]
    x = jnp.max(x.reshape(_B, _M4, x.shape[1]), axis=1)
    out_ref[...] = (jnp.dot(x, lw_ref[...], preferred_element_type=jnp.float32)
                    + lb_ref[...])


def _head_call(f4, p4, layers, linw, linb):
    args = [f4.reshape(_B * _M4, f4.shape[2]), p4.reshape(_B * _M4, 3)]
    for L in layers:
        args += [L['W'], L['b'][None, :]]
    for L in layers:
        args += [jnp.stack([L['g'], L['be']])]
    args += [linw, linb[None, :]]
    return pl.pallas_call(
        _head_kernel,
        out_shape=jax.ShapeDtypeStruct((_B, _OUT), jnp.float32),
    )(*args)


def _sa_level(xin, cent, pos_t, r, layers, mblk, has_feat):
    w1 = layers[0]['W']
    if has_feat:
        wf, wr = w1[:-3], w1[-3:]
    else:
        wf, wr = w1, w1
    g, c1m = _prep_call(xin, cent, wf, wr, layers[0]['b'][None, :], has_feat)
    nidx, nd2 = _topk_call(cent, pos_t, mblk)
    n = g.shape[1]
    gflat = g.reshape(_B * n, g.shape[2])
    gn = jnp.take(gflat, nidx.reshape(-1), axis=0).reshape(
        _B, cent.shape[1], _KP, g.shape[2])
    p1 = jnp.stack([layers[0]['g'], layers[0]['be']])
    p2 = jnp.stack([layers[1]['g'], layers[1]['be']])
    p3 = jnp.stack([layers[2]['g'], layers[2]['be']])
    return _mlp_call(gn, c1m, nd2, layers[1]['W'], layers[1]['b'][None, :],
                     layers[2]['W'], layers[2]['b'][None, :], p1, p2, p3,
                     r, mblk)


def kernel(positions, params):
    pos_t = jnp.transpose(positions, (2, 0, 1))
    s1, s2, s3, s4 = _fps_call(pos_t)
    c1 = jnp.transpose(s1, (1, 2, 0))
    c2 = jnp.transpose(s2, (1, 2, 0))
    c3 = jnp.transpose(s3, (1, 2, 0))
    c4 = jnp.transpose(s4, (1, 2, 0))
    pt0 = jnp.transpose(positions, (0, 2, 1))
    pt1 = jnp.transpose(s1, (1, 0, 2))
    pt2 = jnp.transpose(s2, (1, 0, 2))
    pt3 = jnp.transpose(s3, (1, 0, 2))

    f1 = _sa_level(positions, c1, pt0, 3.0, params['sa1'], 128, False)
    f2 = _sa_level(jnp.concatenate([f1, c1], axis=2), c2, pt1, 4.0,
                   params['sa2'], 205, True)
    f3 = _sa_level(jnp.concatenate([f2, c2], axis=2), c3, pt2, 5.0,
                   params['sa3'], 52, True)
    f4 = _sa_level(jnp.concatenate([f3, c3], axis=2), c4, pt3, 7.0,
                   params['sa4'], 13, True)

    return _head_call(f4, c4, params['last'], params['linW'], params['linb'])


# speedup vs baseline: 8.0281x; 1.0446x over previous
import functools
import numpy as np
import jax
import jax.numpy as jnp
from jax import lax
from jax.experimental import pallas as pl
from jax.experimental.pallas import tpu as pltpu

_B, _N, _OUT, _K = 8, 4096, 256, 25
_KP = 32
_M1, _M2, _M3, _M4 = 820, 205, 52, 13
_BNINV = float(1.0 / np.sqrt(1.0 + 1e-5))
_INF = float(np.inf)


def _fps_stage(x, y, z, m_out):
    b, n = x.shape
    lane = lax.broadcasted_iota(jnp.int32, (b, n), 1)
    col1 = lax.broadcasted_iota(jnp.int32, (b, m_out), 1)

    x0, y0, z0 = x[:, 0:1], y[:, 0:1], z[:, 0:1]
    dx, dy, dz = x - x0, y - y0, z - z0
    d2 = dx * dx + dy * dy + dz * dz

    zero = jnp.zeros((b, m_out), jnp.float32)
    ox = jnp.where(col1 == 0, x0, zero)
    oy = jnp.where(col1 == 0, y0, zero)
    oz = jnp.where(col1 == 0, z0, zero)

    def body(i, state):
        d2, ox, oy, oz = state
        mx = jnp.max(d2, axis=1, keepdims=True)
        far = jnp.min(jnp.where(d2 == mx, lane, n), axis=1, keepdims=True)
        onehot = lane == far
        px = jnp.sum(jnp.where(onehot, x, 0.0), axis=1, keepdims=True)
        py = jnp.sum(jnp.where(onehot, y, 0.0), axis=1, keepdims=True)
        pz = jnp.sum(jnp.where(onehot, z, 0.0), axis=1, keepdims=True)
        ddx, ddy, ddz = x - px, y - py, z - pz
        nd2 = ddx * ddx + ddy * ddy + ddz * ddz
        d2 = jnp.minimum(d2, nd2)
        colmask = col1 == i
        ox = jnp.where(colmask, px, ox)
        oy = jnp.where(colmask, py, oy)
        oz = jnp.where(colmask, pz, oz)
        return d2, ox, oy, oz

    _, ox, oy, oz = lax.fori_loop(1, m_out, body, (d2, ox, oy, oz))
    return ox, oy, oz


def _fps_kernel(pt_ref, s1_ref, s2_ref, s3_ref, s4_ref):
    x, y, z = pt_ref[0], pt_ref[1], pt_ref[2]
    for m, ref in ((_M1, s1_ref), (_M2, s2_ref), (_M3, s3_ref), (_M4, s4_ref)):
        x, y, z = _fps_stage(x, y, z, m)
        ref[0], ref[1], ref[2] = x, y, z


def _fps_call(pos_t):
    f32 = jnp.float32
    return pl.pallas_call(
        _fps_kernel,
        out_shape=(
            jax.ShapeDtypeStruct((3, _B, _M1), f32),
            jax.ShapeDtypeStruct((3, _B, _M2), f32),
            jax.ShapeDtypeStruct((3, _B, _M3), f32),
            jax.ShapeDtypeStruct((3, _B, _M4), f32),
        ),
    )(pos_t)


def _prep_kernel(x_ref, c_ref, wf_ref, wr_ref, b_ref, g_ref, c1_ref, *, has_feat):
    if has_feat:
        feat = x_ref[0][:, :-3]
        posr = x_ref[0][:, -3:]
        g = jnp.dot(feat, wf_ref[...], preferred_element_type=jnp.float32)
        g = g + jnp.dot(posr, wr_ref[...], preferred_element_type=jnp.float32)
    else:
        posr = x_ref[0]
        g = jnp.dot(posr, wr_ref[...], preferred_element_type=jnp.float32)
    g = g + b_ref[...]
    c1 = g.shape[1]
    if c1 < 128:
        g = jnp.concatenate([g, jnp.zeros((g.shape[0], 128 - c1), jnp.float32)],
                            axis=1)
    g_ref[0] = g
    c1_ref[0] = jnp.dot(c_ref[0], wr_ref[...], preferred_element_type=jnp.float32)


def _prep_call(xin, cent, wf, wr, b1, has_feat):
    bb, n, cin = xin.shape
    m = cent.shape[1]
    c1 = wr.shape[1]
    kfn = functools.partial(_prep_kernel, has_feat=has_feat)
    return pl.pallas_call(
        kfn,
        grid=(_B,),
        in_specs=[
            pl.BlockSpec((1, n, cin), lambda b: (b, 0, 0)),
            pl.BlockSpec((1, m, 3), lambda b: (b, 0, 0)),
            pl.BlockSpec(wf.shape, lambda b: (0, 0)),
            pl.BlockSpec(wr.shape, lambda b: (0, 0)),
            pl.BlockSpec(b1.shape, lambda b: (0, 0)),
        ],
        out_specs=(
            pl.BlockSpec((1, n, 128), lambda b: (b, 0, 0)),
            pl.BlockSpec((1, m, c1), lambda b: (b, 0, 0)),
        ),
        out_shape=(
            jax.ShapeDtypeStruct((bb, n, 128), jnp.float32),
            jax.ShapeDtypeStruct((bb, m, c1), jnp.float32),
        ),
    )(xin, cent, wf, wr, b1)


def _topk_kernel(c_ref, pt_ref, idx_ref, nd2_ref, *, n):
    b = pl.program_id(0)
    cb = c_ref[0]
    ptb = pt_ref[0]
    mb = cb.shape[0]
    cp = jnp.dot(cb, ptb, preferred_element_type=jnp.float32)
    csq = jnp.sum(cb * cb, axis=1, keepdims=True)
    psq = jnp.sum(ptb * ptb, axis=0, keepdims=True)
    dd = csq + psq - 2.0 * cp

    lane = lax.broadcasted_iota(jnp.int32, (mb, n), 1)
    colk = lax.broadcasted_iota(jnp.int32, (mb, _KP), 1)
    pool_i = jnp.zeros((mb, _KP), jnp.int32)
    pool_d = jnp.full((mb, _KP), _INF, jnp.float32)

    def body(r, state):
        dd, pool_i, pool_d = state
        mn = jnp.min(dd, axis=1, keepdims=True)
        fi = jnp.min(jnp.where(dd == mn, lane, jnp.int32(2 ** 30)),
                     axis=1, keepdims=True)
        dd = jnp.where(lane == fi, _INF, dd)
        colm = colk == r
        pool_d = jnp.where(colm, mn, pool_d)
        pool_i = jnp.where(colm, fi, pool_i)
        return dd, pool_i, pool_d

    _, pool_i, pool_d = lax.fori_loop(0, _K, body, (dd, pool_i, pool_d))
    idx_ref[0] = pool_i + b * n
    nd2_ref[0] = pool_d


def _topk_call(cent, pos_t, mblk):
    bb, m, _ = cent.shape
    n = pos_t.shape[2]
    nmb = (m + mblk - 1) // mblk
    kfn = functools.partial(_topk_kernel, n=n)
    return pl.pallas_call(
        kfn,
        grid=(_B, nmb),
        in_specs=[
            pl.BlockSpec((1, mblk, 3), lambda b, mi: (b, mi, 0)),
            pl.BlockSpec((1, 3, n), lambda b, mi: (b, 0, 0)),
        ],
        out_specs=(
            pl.BlockSpec((1, mblk, _KP), lambda b, mi: (b, mi, 0)),
            pl.BlockSpec((1, mblk, _KP), lambda b, mi: (b, mi, 0)),
        ),
        out_shape=(
            jax.ShapeDtypeStruct((bb, m, _KP), jnp.int32),
            jax.ShapeDtypeStruct((bb, m, _KP), jnp.float32),
        ),
    )(cent, pos_t)


_NW = 32


def _sc_gather_call(table, idx):
    from jax.experimental.pallas import tpu_sc as plsc

    bt = idx.shape[0]
    d = table.shape[1]
    step = 128
    s = -(-bt // (_NW * step))
    bpad = _NW * s * step
    if bpad != bt:
        idx = jnp.concatenate([idx, jnp.zeros((bpad - bt,), jnp.int32)])
    idx3 = idx.reshape(_NW, s, step)

    mesh = plsc.VectorSubcoreMesh(core_axis_name="c", subcore_axis_name="s")

    @functools.partial(
        pl.kernel, mesh=mesh,
        out_type=jax.ShapeDtypeStruct((bpad, d), jnp.float32),
        scratch_types=[
            pltpu.VMEM((s, step), jnp.int32),
            pltpu.VMEM((step, d), jnp.float32),
            pltpu.SemaphoreType.DMA,
        ],
    )
    def gk(table_hbm, idx_hbm, out_hbm, idx_v, rows_v, sem):
        wid = lax.axis_index("s") * 2 + lax.axis_index("c")
        pltpu.sync_copy(idx_hbm.at[wid], idx_v)
        base = wid * (s * step)
        for st in range(s):
            pltpu.async_copy(table_hbm.at[idx_v.at[st]], rows_v, sem).wait()
            pltpu.sync_copy(rows_v, out_hbm.at[pl.ds(base + st * step, step)])

    out = gk(table, idx3)
    return out[:bt] if bpad != bt else out


def _mlp_kernel(gn_ref, c1_ref, nd2_ref, w2_ref, b2_ref, w3_ref, b3_ref,
                p1_ref, p2_ref, p3_ref, out_ref, *, r2):
    mb = c1_ref.shape[1]
    c1w = c1_ref.shape[2]
    gn = gn_ref[0][:, :, :c1w]
    pre = gn - c1_ref[0][:, None, :]
    h = jnp.maximum(pre, 0.0).reshape(mb * _KP, c1w)
    g1, be1 = p1_ref[0:1], p1_ref[1:2]
    h = (h * _BNINV) * g1 + be1
    h = jnp.maximum(jnp.dot(h, w2_ref[...], preferred_element_type=jnp.float32)
                    + b2_ref[...], 0.0)
    h = (h * _BNINV) * p2_ref[0:1] + p2_ref[1:2]
    h = jnp.maximum(jnp.dot(h, w3_ref[...], preferred_element_type=jnp.float32)
                    + b3_ref[...], 0.0)
    h = (h * _BNINV) * p3_ref[0:1] + p3_ref[1:2]
    cout = h.shape[1]
    h = h.reshape(mb, _KP, cout)
    maskadd = jnp.where(nd2_ref[0] <= r2, 0.0, -_INF)
    h = h + maskadd
    out_ref[0] = jnp.max(h, axis=1)


def _mlp_call(gn, c1m, nd2, w2, b2, w3, b3, p1, p2, p3, r, mblk):
    bb, m = gn.shape[0], gn.shape[1]
    gd = gn.shape[3]
    cout = w3.shape[1]
    nmb = (m + mblk - 1) // mblk
    kfn = functools.partial(_mlp_kernel, r2=float(r * r))
    return pl.pallas_call(
        kfn,
        grid=(_B, nmb),
        in_specs=[
            pl.BlockSpec((1, mblk, _KP, gd), lambda b, mi: (b, mi, 0, 0)),
            pl.BlockSpec((1, mblk, c1m.shape[2]), lambda b, mi: (b, mi, 0)),
            pl.BlockSpec((1, mblk, _KP, 1), lambda b, mi: (b, mi, 0, 0)),
            pl.BlockSpec(w2.shape, lambda b, mi: (0, 0)),
            pl.BlockSpec(b2.shape, lambda b, mi: (0, 0)),
            pl.BlockSpec(w3.shape, lambda b, mi: (0, 0)),
            pl.BlockSpec(b3.shape, lambda b, mi: (0, 0)),
            pl.BlockSpec(p1.shape, lambda b, mi: (0, 0)),
            pl.BlockSpec(p2.shape, lambda b, mi: (0, 0)),
            pl.BlockSpec(p3.shape, lambda b, mi: (0, 0)),
        ],
        out_specs=pl.BlockSpec((1, mblk, cout), lambda b, mi: (b, mi, 0)),
        out_shape=jax.ShapeDtypeStruct((bb, m, cout), jnp.float32),
    )(gn, c1m, nd2.reshape(bb, m, _KP, 1), w2, b2, w3, b3, p1, p2, p3)


def _head_kernel(f_ref, p_ref, w1_ref, b1_ref, w2_ref, b2_ref, w3_ref, b3_ref,
                 g1_ref, g2_ref, g3_ref, lw_ref, lb_ref, out_ref):
    x = jnp.concatenate([f_ref[...], p_ref[...]], axis=1)
    for w, b, p in ((w1_ref, b1_ref, g1_ref), (w2_ref, b2_ref, g2_ref),
                    (w3_ref, b3_ref, g3_ref)):
        x = jnp.maximum(jnp.dot(x, w[...], preferred_element_type=jnp.float32)
                        + b[...], 0.0)
        x = (x * _BNINV) * p[0:1] + p[1:2]
    x = jnp.max(x.reshape(_B, _M4, x.shape[1]), axis=1)
    out_ref[...] = (jnp.dot(x, lw_ref[...], preferred_element_type=jnp.float32)
                    + lb_ref[...])


def _head_call(f4, p4, layers, linw, linb):
    args = [f4.reshape(_B * _M4, f4.shape[2]), p4.reshape(_B * _M4, 3)]
    for L in layers:
        args += [L['W'], L['b'][None, :]]
    for L in layers:
        args += [jnp.stack([L['g'], L['be']])]
    args += [linw, linb[None, :]]
    return pl.pallas_call(
        _head_kernel,
        out_shape=jax.ShapeDtypeStruct((_B, _OUT), jnp.float32),
    )(*args)


def _sa_level(xin, cent, pos_t, r, layers, mblk, has_feat):
    w1 = layers[0]['W']
    if has_feat:
        wf, wr = w1[:-3], w1[-3:]
    else:
        wf, wr = w1, w1
    g, c1m = _prep_call(xin, cent, wf, wr, layers[0]['b'][None, :], has_feat)
    nidx, nd2 = _topk_call(cent, pos_t, mblk)
    n = g.shape[1]
    gflat = g.reshape(_B * n, g.shape[2])
    gn = _sc_gather_call(gflat, nidx.reshape(-1)).reshape(
        _B, cent.shape[1], _KP, g.shape[2])
    p1 = jnp.stack([layers[0]['g'], layers[0]['be']])
    p2 = jnp.stack([layers[1]['g'], layers[1]['be']])
    p3 = jnp.stack([layers[2]['g'], layers[2]['be']])
    return _mlp_call(gn, c1m, nd2, layers[1]['W'], layers[1]['b'][None, :],
                     layers[2]['W'], layers[2]['b'][None, :], p1, p2, p3,
                     r, mblk)


def kernel(positions, params):
    pos_t = jnp.transpose(positions, (2, 0, 1))
    s1, s2, s3, s4 = _fps_call(pos_t)
    c1 = jnp.transpose(s1, (1, 2, 0))
    c2 = jnp.transpose(s2, (1, 2, 0))
    c3 = jnp.transpose(s3, (1, 2, 0))
    c4 = jnp.transpose(s4, (1, 2, 0))
    pt0 = jnp.transpose(positions, (0, 2, 1))
    pt1 = jnp.transpose(s1, (1, 0, 2))
    pt2 = jnp.transpose(s2, (1, 0, 2))
    pt3 = jnp.transpose(s3, (1, 0, 2))

    f1 = _sa_level(positions, c1, pt0, 3.0, params['sa1'], 128, False)
    f2 = _sa_level(jnp.concatenate([f1, c1], axis=2), c2, pt1, 4.0,
                   params['sa2'], 205, True)
    f3 = _sa_level(jnp.concatenate([f2, c2], axis=2), c3, pt2, 5.0,
                   params['sa3'], 52, True)
    f4 = _sa_level(jnp.concatenate([f3, c3], axis=2), c4, pt3, 7.0,
                   params['sa4'], 13, True)

    return _head_call(f4, c4, params['last'], params['linW'], params['linb'])


# speedup vs baseline: 10.4365x; 1.3000x over previous
import functools
import numpy as np
import jax
import jax.numpy as jnp
from jax import lax
from jax.experimental import pallas as pl
from jax.experimental.pallas import tpu as pltpu

_B, _N, _OUT, _K = 8, 4096, 256, 25
_KP = 32
_M1, _M2, _M3, _M4 = 820, 205, 52, 13
_BNINV = float(1.0 / np.sqrt(1.0 + 1e-5))
_INF = float(np.inf)


def _fps_stage(x, y, z, m_out):
    b, n = x.shape
    lane = lax.broadcasted_iota(jnp.int32, (b, n), 1)
    col1 = lax.broadcasted_iota(jnp.int32, (b, m_out), 1)

    x0, y0, z0 = x[:, 0:1], y[:, 0:1], z[:, 0:1]
    dx, dy, dz = x - x0, y - y0, z - z0
    d2 = dx * dx + dy * dy + dz * dz

    zero = jnp.zeros((b, m_out), jnp.float32)
    ox = jnp.where(col1 == 0, x0, zero)
    oy = jnp.where(col1 == 0, y0, zero)
    oz = jnp.where(col1 == 0, z0, zero)

    def body(i, state):
        d2, ox, oy, oz = state
        mx = jnp.max(d2, axis=1, keepdims=True)
        far = jnp.min(jnp.where(d2 == mx, lane, n), axis=1, keepdims=True)
        onehot = lane == far
        px = jnp.sum(jnp.where(onehot, x, 0.0), axis=1, keepdims=True)
        py = jnp.sum(jnp.where(onehot, y, 0.0), axis=1, keepdims=True)
        pz = jnp.sum(jnp.where(onehot, z, 0.0), axis=1, keepdims=True)
        ddx, ddy, ddz = x - px, y - py, z - pz
        nd2 = ddx * ddx + ddy * ddy + ddz * ddz
        d2 = jnp.minimum(d2, nd2)
        colmask = col1 == i
        ox = jnp.where(colmask, px, ox)
        oy = jnp.where(colmask, py, oy)
        oz = jnp.where(colmask, pz, oz)
        return d2, ox, oy, oz

    _, ox, oy, oz = lax.fori_loop(1, m_out, body, (d2, ox, oy, oz))
    return ox, oy, oz


def _fps_kernel(pt_ref, s1_ref, s2_ref, s3_ref, s4_ref):
    x, y, z = pt_ref[0], pt_ref[1], pt_ref[2]
    for m, ref in ((_M1, s1_ref), (_M2, s2_ref), (_M3, s3_ref), (_M4, s4_ref)):
        x, y, z = _fps_stage(x, y, z, m)
        ref[0], ref[1], ref[2] = x, y, z


def _fps_call(pos_t):
    f32 = jnp.float32
    return pl.pallas_call(
        _fps_kernel,
        out_shape=(
            jax.ShapeDtypeStruct((3, _B, _M1), f32),
            jax.ShapeDtypeStruct((3, _B, _M2), f32),
            jax.ShapeDtypeStruct((3, _B, _M3), f32),
            jax.ShapeDtypeStruct((3, _B, _M4), f32),
        ),
    )(pos_t)


def _prep_kernel(x_ref, c_ref, wf_ref, wr_ref, b_ref, g_ref, c1_ref, *, has_feat):
    if has_feat:
        feat = x_ref[0][:, :-3]
        posr = x_ref[0][:, -3:]
        g = jnp.dot(feat, wf_ref[...], preferred_element_type=jnp.float32)
        g = g + jnp.dot(posr, wr_ref[...], preferred_element_type=jnp.float32)
    else:
        posr = x_ref[0]
        g = jnp.dot(posr, wr_ref[...], preferred_element_type=jnp.float32)
    g = g + b_ref[...]
    c1 = g.shape[1]
    if c1 < 128:
        g = jnp.concatenate([g, jnp.zeros((g.shape[0], 128 - c1), jnp.float32)],
                            axis=1)
    g_ref[0] = g
    c1_ref[0] = jnp.dot(c_ref[0], wr_ref[...], preferred_element_type=jnp.float32)


def _prep_call(xin, cent, wf, wr, b1, has_feat):
    bb, n, cin = xin.shape
    m = cent.shape[1]
    c1 = wr.shape[1]
    kfn = functools.partial(_prep_kernel, has_feat=has_feat)
    return pl.pallas_call(
        kfn,
        grid=(_B,),
        in_specs=[
            pl.BlockSpec((1, n, cin), lambda b: (b, 0, 0)),
            pl.BlockSpec((1, m, 3), lambda b: (b, 0, 0)),
            pl.BlockSpec(wf.shape, lambda b: (0, 0)),
            pl.BlockSpec(wr.shape, lambda b: (0, 0)),
            pl.BlockSpec(b1.shape, lambda b: (0, 0)),
        ],
        out_specs=(
            pl.BlockSpec((1, n, 128), lambda b: (b, 0, 0)),
            pl.BlockSpec((1, m, c1), lambda b: (b, 0, 0)),
        ),
        out_shape=(
            jax.ShapeDtypeStruct((bb, n, 128), jnp.float32),
            jax.ShapeDtypeStruct((bb, m, c1), jnp.float32),
        ),
    )(xin, cent, wf, wr, b1)


def _topk_kernel(c_ref, pt_ref, idx_ref, nd2_ref, *, n):
    b = pl.program_id(0)
    cb = c_ref[0]
    ptb = pt_ref[0]
    mb = cb.shape[0]
    cp = jnp.dot(cb, ptb, preferred_element_type=jnp.float32)
    csq = jnp.sum(cb * cb, axis=1, keepdims=True)
    psq = jnp.sum(ptb * ptb, axis=0, keepdims=True)
    dd = csq + psq - 2.0 * cp

    lane = lax.broadcasted_iota(jnp.int32, (mb, n), 1)
    colk = lax.broadcasted_iota(jnp.int32, (mb, _KP), 1)
    pool_i = jnp.zeros((mb, _KP), jnp.int32)
    pool_d = jnp.full((mb, _KP), _INF, jnp.float32)

    for r in range(_K):
        mn = jnp.min(dd, axis=1, keepdims=True)
        fi = jnp.min(jnp.where(dd == mn, lane, jnp.int32(2 ** 30)),
                     axis=1, keepdims=True)
        dd = jnp.where(lane == fi, _INF, dd)
        colm = colk == r
        pool_d = jnp.where(colm, mn, pool_d)
        pool_i = jnp.where(colm, fi, pool_i)
    idx_ref[0] = pool_i + b * n
    nd2_ref[0] = pool_d


def _topk_call(cent, pos_t, mblk):
    bb, m, _ = cent.shape
    n = pos_t.shape[2]
    nmb = (m + mblk - 1) // mblk
    kfn = functools.partial(_topk_kernel, n=n)
    return pl.pallas_call(
        kfn,
        grid=(_B, nmb),
        in_specs=[
            pl.BlockSpec((1, mblk, 3), lambda b, mi: (b, mi, 0)),
            pl.BlockSpec((1, 3, n), lambda b, mi: (b, 0, 0)),
        ],
        out_specs=(
            pl.BlockSpec((1, mblk, _KP), lambda b, mi: (b, mi, 0)),
            pl.BlockSpec((1, mblk, _KP), lambda b, mi: (b, mi, 0)),
        ),
        out_shape=(
            jax.ShapeDtypeStruct((bb, m, _KP), jnp.int32),
            jax.ShapeDtypeStruct((bb, m, _KP), jnp.float32),
        ),
    )(cent, pos_t)


_NW = 32


def _sc_gather_call(table, idx):
    from jax.experimental.pallas import tpu_sc as plsc

    bt = idx.shape[0]
    d = table.shape[1]
    step = 128
    s = -(-bt // (_NW * step))
    bpad = _NW * s * step
    if bpad != bt:
        idx = jnp.concatenate([idx, jnp.zeros((bpad - bt,), jnp.int32)])
    idx3 = idx.reshape(_NW, s, step)

    mesh = plsc.VectorSubcoreMesh(core_axis_name="c", subcore_axis_name="s")

    @functools.partial(
        pl.kernel, mesh=mesh,
        out_type=jax.ShapeDtypeStruct((bpad, d), jnp.float32),
        scratch_types=[
            pltpu.VMEM((s, step), jnp.int32),
            pltpu.VMEM((step, d), jnp.float32),
            pltpu.SemaphoreType.DMA,
        ],
    )
    def gk(table_hbm, idx_hbm, out_hbm, idx_v, rows_v, sem):
        wid = lax.axis_index("s") * 2 + lax.axis_index("c")
        pltpu.sync_copy(idx_hbm.at[wid], idx_v)
        base = wid * (s * step)
        for st in range(s):
            pltpu.async_copy(table_hbm.at[idx_v.at[st]], rows_v, sem).wait()
            pltpu.sync_copy(rows_v, out_hbm.at[pl.ds(base + st * step, step)])

    out = gk(table, idx3)
    return out[:bt] if bpad != bt else out


def _mlp_kernel(gn_ref, c1_ref, nd2_ref, w2_ref, b2_ref, w3_ref, b3_ref,
                p1_ref, p2_ref, p3_ref, out_ref, *, r2):
    mb = c1_ref.shape[1]
    c1w = c1_ref.shape[2]
    gn = gn_ref[0][:, :, :c1w]
    pre = gn - c1_ref[0][:, None, :]
    h = jnp.maximum(pre, 0.0).reshape(mb * _KP, c1w)
    g1, be1 = p1_ref[0:1], p1_ref[1:2]
    h = (h * _BNINV) * g1 + be1
    h = jnp.maximum(jnp.dot(h, w2_ref[...], preferred_element_type=jnp.float32)
                    + b2_ref[...], 0.0)
    h = (h * _BNINV) * p2_ref[0:1] + p2_ref[1:2]
    h = jnp.maximum(jnp.dot(h, w3_ref[...], preferred_element_type=jnp.float32)
                    + b3_ref[...], 0.0)
    h = (h * _BNINV) * p3_ref[0:1] + p3_ref[1:2]
    cout = h.shape[1]
    h = h.reshape(mb, _KP, cout)
    maskadd = jnp.where(nd2_ref[0] <= r2, 0.0, -_INF)
    h = h + maskadd
    out_ref[0] = jnp.max(h, axis=1)


def _mlp_call(gn, c1m, nd2, w2, b2, w3, b3, p1, p2, p3, r, mblk):
    bb, m = gn.shape[0], gn.shape[1]
    gd = gn.shape[3]
    cout = w3.shape[1]
    nmb = (m + mblk - 1) // mblk
    kfn = functools.partial(_mlp_kernel, r2=float(r * r))
    return pl.pallas_call(
        kfn,
        grid=(_B, nmb),
        in_specs=[
            pl.BlockSpec((1, mblk, _KP, gd), lambda b, mi: (b, mi, 0, 0)),
            pl.BlockSpec((1, mblk, c1m.shape[2]), lambda b, mi: (b, mi, 0)),
            pl.BlockSpec((1, mblk, _KP, 1), lambda b, mi: (b, mi, 0, 0)),
            pl.BlockSpec(w2.shape, lambda b, mi: (0, 0)),
            pl.BlockSpec(b2.shape, lambda b, mi: (0, 0)),
            pl.BlockSpec(w3.shape, lambda b, mi: (0, 0)),
            pl.BlockSpec(b3.shape, lambda b, mi: (0, 0)),
            pl.BlockSpec(p1.shape, lambda b, mi: (0, 0)),
            pl.BlockSpec(p2.shape, lambda b, mi: (0, 0)),
            pl.BlockSpec(p3.shape, lambda b, mi: (0, 0)),
        ],
        out_specs=pl.BlockSpec((1, mblk, cout), lambda b, mi: (b, mi, 0)),
        out_shape=jax.ShapeDtypeStruct((bb, m, cout), jnp.float32),
    )(gn, c1m, nd2.reshape(bb, m, _KP, 1), w2, b2, w3, b3, p1, p2, p3)


def _head_kernel(f_ref, p_ref, w1_ref, b1_ref, w2_ref, b2_ref, w3_ref, b3_ref,
                 g1_ref, g2_ref, g3_ref, lw_ref, lb_ref, out_ref):
    x = jnp.concatenate([f_ref[...], p_ref[...]], axis=1)
    for w, b, p in ((w1_ref, b1_ref, g1_ref), (w2_ref, b2_ref, g2_ref),
                    (w3_ref, b3_ref, g3_ref)):
        x = jnp.maximum(jnp.dot(x, w[...], preferred_element_type=jnp.float32)
                        + b[...], 0.0)
        x = (x * _BNINV) * p[0:1] + p[1:2]
    x = jnp.max(x.reshape(_B, _M4, x.shape[1]), axis=1)
    out_ref[...] = (jnp.dot(x, lw_ref[...], preferred_element_type=jnp.float32)
                    + lb_ref[...])


def _head_call(f4, p4, layers, linw, linb):
    args = [f4.reshape(_B * _M4, f4.shape[2]), p4.reshape(_B * _M4, 3)]
    for L in layers:
        args += [L['W'], L['b'][None, :]]
    for L in layers:
        args += [jnp.stack([L['g'], L['be']])]
    args += [linw, linb[None, :]]
    return pl.pallas_call(
        _head_kernel,
        out_shape=jax.ShapeDtypeStruct((_B, _OUT), jnp.float32),
    )(*args)


def _sa_level(xin, cent, pos_t, r, layers, mblk, has_feat):
    w1 = layers[0]['W']
    if has_feat:
        wf, wr = w1[:-3], w1[-3:]
    else:
        wf, wr = w1, w1
    g, c1m = _prep_call(xin, cent, wf, wr, layers[0]['b'][None, :], has_feat)
    nidx, nd2 = _topk_call(cent, pos_t, mblk)
    n = g.shape[1]
    gflat = g.reshape(_B * n, g.shape[2])
    gn = _sc_gather_call(gflat, nidx.reshape(-1)).reshape(
        _B, cent.shape[1], _KP, g.shape[2])
    p1 = jnp.stack([layers[0]['g'], layers[0]['be']])
    p2 = jnp.stack([layers[1]['g'], layers[1]['be']])
    p3 = jnp.stack([layers[2]['g'], layers[2]['be']])
    return _mlp_call(gn, c1m, nd2, layers[1]['W'], layers[1]['b'][None, :],
                     layers[2]['W'], layers[2]['b'][None, :], p1, p2, p3,
                     r, mblk)


def kernel(positions, params):
    pos_t = jnp.transpose(positions, (2, 0, 1))
    s1, s2, s3, s4 = _fps_call(pos_t)
    c1 = jnp.transpose(s1, (1, 2, 0))
    c2 = jnp.transpose(s2, (1, 2, 0))
    c3 = jnp.transpose(s3, (1, 2, 0))
    c4 = jnp.transpose(s4, (1, 2, 0))
    pt0 = jnp.transpose(positions, (0, 2, 1))
    pt1 = jnp.transpose(s1, (1, 0, 2))
    pt2 = jnp.transpose(s2, (1, 0, 2))
    pt3 = jnp.transpose(s3, (1, 0, 2))

    f1 = _sa_level(positions, c1, pt0, 3.0, params['sa1'], 208, False)
    f2 = _sa_level(jnp.concatenate([f1, c1], axis=2), c2, pt1, 4.0,
                   params['sa2'], 205, True)
    f3 = _sa_level(jnp.concatenate([f2, c2], axis=2), c3, pt2, 5.0,
                   params['sa3'], 52, True)
    f4 = _sa_level(jnp.concatenate([f3, c3], axis=2), c4, pt3, 7.0,
                   params['sa4'], 13, True)

    return _head_call(f4, c4, params['last'], params['linW'], params['linb'])


# speedup vs baseline: 10.4495x; 1.0012x over previous
import functools
import numpy as np
import jax
import jax.numpy as jnp
from jax import lax
from jax.experimental import pallas as pl
from jax.experimental.pallas import tpu as pltpu

_B, _N, _OUT, _K = 8, 4096, 256, 25
_KP = 32
_M1, _M2, _M3, _M4 = 820, 205, 52, 13
_BNINV = float(1.0 / np.sqrt(1.0 + 1e-5))
_INF = float(np.inf)


def _fps_stage(x, y, z, m_out):
    b, n = x.shape
    lane = lax.broadcasted_iota(jnp.int32, (b, n), 1)
    col1 = lax.broadcasted_iota(jnp.int32, (b, m_out), 1)

    x0, y0, z0 = x[:, 0:1], y[:, 0:1], z[:, 0:1]
    dx, dy, dz = x - x0, y - y0, z - z0
    d2 = dx * dx + dy * dy + dz * dz

    zero = jnp.zeros((b, m_out), jnp.float32)
    ox = jnp.where(col1 == 0, x0, zero)
    oy = jnp.where(col1 == 0, y0, zero)
    oz = jnp.where(col1 == 0, z0, zero)

    def body(i, state):
        d2, ox, oy, oz = state
        mx = jnp.max(d2, axis=1, keepdims=True)
        far = jnp.min(jnp.where(d2 == mx, lane, n), axis=1, keepdims=True)
        onehot = lane == far
        px = jnp.sum(jnp.where(onehot, x, 0.0), axis=1, keepdims=True)
        py = jnp.sum(jnp.where(onehot, y, 0.0), axis=1, keepdims=True)
        pz = jnp.sum(jnp.where(onehot, z, 0.0), axis=1, keepdims=True)
        ddx, ddy, ddz = x - px, y - py, z - pz
        nd2 = ddx * ddx + ddy * ddy + ddz * ddz
        d2 = jnp.minimum(d2, nd2)
        colmask = col1 == i
        ox = jnp.where(colmask, px, ox)
        oy = jnp.where(colmask, py, oy)
        oz = jnp.where(colmask, pz, oz)
        return d2, ox, oy, oz

    _, ox, oy, oz = lax.fori_loop(1, m_out, body, (d2, ox, oy, oz))
    return ox, oy, oz


def _fps_kernel(pt_ref, s1_ref, s2_ref, s3_ref, s4_ref):
    x, y, z = pt_ref[0], pt_ref[1], pt_ref[2]
    for m, ref in ((_M1, s1_ref), (_M2, s2_ref), (_M3, s3_ref), (_M4, s4_ref)):
        x, y, z = _fps_stage(x, y, z, m)
        ref[0], ref[1], ref[2] = x, y, z


def _fps_call(pos_t):
    f32 = jnp.float32
    return pl.pallas_call(
        _fps_kernel,
        out_shape=(
            jax.ShapeDtypeStruct((3, _B, _M1), f32),
            jax.ShapeDtypeStruct((3, _B, _M2), f32),
            jax.ShapeDtypeStruct((3, _B, _M3), f32),
            jax.ShapeDtypeStruct((3, _B, _M4), f32),
        ),
    )(pos_t)


def _prep_kernel(x_ref, c_ref, wf_ref, wr_ref, b_ref, g_ref, c1_ref, *, has_feat):
    if has_feat:
        feat = x_ref[0][:, :-3]
        posr = x_ref[0][:, -3:]
        g = jnp.dot(feat, wf_ref[...], preferred_element_type=jnp.float32)
        g = g + jnp.dot(posr, wr_ref[...], preferred_element_type=jnp.float32)
    else:
        posr = x_ref[0]
        g = jnp.dot(posr, wr_ref[...], preferred_element_type=jnp.float32)
    g = g + b_ref[...]
    c1 = g.shape[1]
    if c1 < 128:
        g = jnp.concatenate([g, jnp.zeros((g.shape[0], 128 - c1), jnp.float32)],
                            axis=1)
    g_ref[0] = g
    c1_ref[0] = jnp.dot(c_ref[0], wr_ref[...], preferred_element_type=jnp.float32)


def _prep_call(xin, cent, wf, wr, b1, has_feat):
    bb, n, cin = xin.shape
    m = cent.shape[1]
    c1 = wr.shape[1]
    kfn = functools.partial(_prep_kernel, has_feat=has_feat)
    return pl.pallas_call(
        kfn,
        grid=(_B,),
        in_specs=[
            pl.BlockSpec((1, n, cin), lambda b: (b, 0, 0)),
            pl.BlockSpec((1, m, 3), lambda b: (b, 0, 0)),
            pl.BlockSpec(wf.shape, lambda b: (0, 0)),
            pl.BlockSpec(wr.shape, lambda b: (0, 0)),
            pl.BlockSpec(b1.shape, lambda b: (0, 0)),
        ],
        out_specs=(
            pl.BlockSpec((1, n, 128), lambda b: (b, 0, 0)),
            pl.BlockSpec((1, m, c1), lambda b: (b, 0, 0)),
        ),
        out_shape=(
            jax.ShapeDtypeStruct((bb, n, 128), jnp.float32),
            jax.ShapeDtypeStruct((bb, m, c1), jnp.float32),
        ),
    )(xin, cent, wf, wr, b1)


def _topk_kernel(c_ref, pt_ref, idx_ref, nd2_ref, *, n):
    b = pl.program_id(0)
    cb = c_ref[0]
    ptb = pt_ref[0]
    mb = cb.shape[0]
    cp = jnp.dot(cb, ptb, preferred_element_type=jnp.float32)
    csq = jnp.sum(cb * cb, axis=1, keepdims=True)
    psq = jnp.sum(ptb * ptb, axis=0, keepdims=True)
    dd = csq + psq - 2.0 * cp

    lane = lax.broadcasted_iota(jnp.int32, (mb, n), 1)
    colk = lax.broadcasted_iota(jnp.int32, (mb, _KP), 1)
    pool_i = jnp.zeros((mb, _KP), jnp.int32)
    pool_d = jnp.full((mb, _KP), _INF, jnp.float32)

    for r in range(_K):
        mn = jnp.min(dd, axis=1, keepdims=True)
        fi = jnp.min(jnp.where(dd == mn, lane, jnp.int32(2 ** 30)),
                     axis=1, keepdims=True)
        dd = jnp.where(lane == fi, _INF, dd)
        colm = colk == r
        pool_d = jnp.where(colm, mn, pool_d)
        pool_i = jnp.where(colm, fi, pool_i)
    idx_ref[0] = pool_i + b * n
    nd2_ref[0] = pool_d


def _topk_call(cent, pos_t, mblk):
    bb, m, _ = cent.shape
    n = pos_t.shape[2]
    nmb = (m + mblk - 1) // mblk
    kfn = functools.partial(_topk_kernel, n=n)
    return pl.pallas_call(
        kfn,
        grid=(_B, nmb),
        in_specs=[
            pl.BlockSpec((1, mblk, 3), lambda b, mi: (b, mi, 0)),
            pl.BlockSpec((1, 3, n), lambda b, mi: (b, 0, 0)),
        ],
        out_specs=(
            pl.BlockSpec((1, mblk, _KP), lambda b, mi: (b, mi, 0)),
            pl.BlockSpec((1, mblk, _KP), lambda b, mi: (b, mi, 0)),
        ),
        out_shape=(
            jax.ShapeDtypeStruct((bb, m, _KP), jnp.int32),
            jax.ShapeDtypeStruct((bb, m, _KP), jnp.float32),
        ),
    )(cent, pos_t)


_NW = 32


def _sc_gather_call(table, idx):
    from jax.experimental.pallas import tpu_sc as plsc

    bt = idx.shape[0]
    d = table.shape[1]
    step = 128
    s = -(-bt // (_NW * step))
    bpad = _NW * s * step
    if bpad != bt:
        idx = jnp.concatenate([idx, jnp.zeros((bpad - bt,), jnp.int32)])
    idx3 = idx.reshape(_NW, s, step)

    mesh = plsc.VectorSubcoreMesh(core_axis_name="c", subcore_axis_name="s")

    nbuf = min(4, s)

    @functools.partial(
        pl.kernel, mesh=mesh,
        out_type=jax.ShapeDtypeStruct((bpad, d), jnp.float32),
        scratch_types=[
            pltpu.VMEM((s, step), jnp.int32),
            pltpu.VMEM((nbuf, step, d), jnp.float32),
            pltpu.SemaphoreType.DMA,
            pltpu.SemaphoreType.DMA,
        ],
    )
    def gk(table_hbm, idx_hbm, out_hbm, idx_v, rows_v, gsem, osem):
        wid = lax.axis_index("s") * 2 + lax.axis_index("c")
        pltpu.sync_copy(idx_hbm.at[wid], idx_v)
        base = wid * (s * step)
        puts = []
        for g0 in range(0, s, nbuf):
            cnt = min(nbuf, s - g0)
            for p in puts:
                p.wait()
            gets = []
            for j in range(cnt):
                gets.append(pltpu.async_copy(
                    table_hbm.at[idx_v.at[g0 + j]], rows_v.at[j], gsem))
            puts = []
            for j in range(cnt):
                gets[j].wait()
                puts.append(pltpu.async_copy(
                    rows_v.at[j],
                    out_hbm.at[pl.ds(base + (g0 + j) * step, step)], osem))
        for p in puts:
            p.wait()

    out = gk(table, idx3)
    return out[:bt] if bpad != bt else out


def _mlp_kernel(gn_ref, c1_ref, nd2_ref, w2_ref, b2_ref, w3_ref, b3_ref,
                p1_ref, p2_ref, p3_ref, out_ref, *, r2):
    mb = c1_ref.shape[1]
    c1w = c1_ref.shape[2]
    gn = gn_ref[0][:, :, :c1w]
    pre = gn - c1_ref[0][:, None, :]
    h = jnp.maximum(pre, 0.0).reshape(mb * _KP, c1w)
    g1, be1 = p1_ref[0:1], p1_ref[1:2]
    h = (h * _BNINV) * g1 + be1
    h = jnp.maximum(jnp.dot(h, w2_ref[...], preferred_element_type=jnp.float32)
                    + b2_ref[...], 0.0)
    h = (h * _BNINV) * p2_ref[0:1] + p2_ref[1:2]
    h = jnp.maximum(jnp.dot(h, w3_ref[...], preferred_element_type=jnp.float32)
                    + b3_ref[...], 0.0)
    h = (h * _BNINV) * p3_ref[0:1] + p3_ref[1:2]
    cout = h.shape[1]
    h = h.reshape(mb, _KP, cout)
    maskadd = jnp.where(nd2_ref[0] <= r2, 0.0, -_INF)
    h = h + maskadd
    out_ref[0] = jnp.max(h, axis=1)


def _mlp_call(gn, c1m, nd2, w2, b2, w3, b3, p1, p2, p3, r, mblk):
    bb, m = gn.shape[0], gn.shape[1]
    gd = gn.shape[3]
    cout = w3.shape[1]
    nmb = (m + mblk - 1) // mblk
    kfn = functools.partial(_mlp_kernel, r2=float(r * r))
    return pl.pallas_call(
        kfn,
        grid=(_B, nmb),
        in_specs=[
            pl.BlockSpec((1, mblk, _KP, gd), lambda b, mi: (b, mi, 0, 0)),
            pl.BlockSpec((1, mblk, c1m.shape[2]), lambda b, mi: (b, mi, 0)),
            pl.BlockSpec((1, mblk, _KP, 1), lambda b, mi: (b, mi, 0, 0)),
            pl.BlockSpec(w2.shape, lambda b, mi: (0, 0)),
            pl.BlockSpec(b2.shape, lambda b, mi: (0, 0)),
            pl.BlockSpec(w3.shape, lambda b, mi: (0, 0)),
            pl.BlockSpec(b3.shape, lambda b, mi: (0, 0)),
            pl.BlockSpec(p1.shape, lambda b, mi: (0, 0)),
            pl.BlockSpec(p2.shape, lambda b, mi: (0, 0)),
            pl.BlockSpec(p3.shape, lambda b, mi: (0, 0)),
        ],
        out_specs=pl.BlockSpec((1, mblk, cout), lambda b, mi: (b, mi, 0)),
        out_shape=jax.ShapeDtypeStruct((bb, m, cout), jnp.float32),
    )(gn, c1m, nd2.reshape(bb, m, _KP, 1), w2, b2, w3, b3, p1, p2, p3)


def _head_kernel(f_ref, p_ref, w1_ref, b1_ref, w2_ref, b2_ref, w3_ref, b3_ref,
                 g1_ref, g2_ref, g3_ref, lw_ref, lb_ref, out_ref):
    x = jnp.concatenate([f_ref[...], p_ref[...]], axis=1)
    for w, b, p in ((w1_ref, b1_ref, g1_ref), (w2_ref, b2_ref, g2_ref),
                    (w3_ref, b3_ref, g3_ref)):
        x = jnp.maximum(jnp.dot(x, w[...], preferred_element_type=jnp.float32)
                        + b[...], 0.0)
        x = (x * _BNINV) * p[0:1] + p[1:2]
    x = jnp.max(x.reshape(_B, _M4, x.shape[1]), axis=1)
    out_ref[...] = (jnp.dot(x, lw_ref[...], preferred_element_type=jnp.float32)
                    + lb_ref[...])


def _head_call(f4, p4, layers, linw, linb):
    args = [f4.reshape(_B * _M4, f4.shape[2]), p4.reshape(_B * _M4, 3)]
    for L in layers:
        args += [L['W'], L['b'][None, :]]
    for L in layers:
        args += [jnp.stack([L['g'], L['be']])]
    args += [linw, linb[None, :]]
    return pl.pallas_call(
        _head_kernel,
        out_shape=jax.ShapeDtypeStruct((_B, _OUT), jnp.float32),
    )(*args)


def _sa_level(xin, cent, pos_t, r, layers, mblk, has_feat):
    w1 = layers[0]['W']
    if has_feat:
        wf, wr = w1[:-3], w1[-3:]
    else:
        wf, wr = w1, w1
    g, c1m = _prep_call(xin, cent, wf, wr, layers[0]['b'][None, :], has_feat)
    nidx, nd2 = _topk_call(cent, pos_t, mblk)
    n = g.shape[1]
    gflat = g.reshape(_B * n, g.shape[2])
    gn = _sc_gather_call(gflat, nidx.reshape(-1)).reshape(
        _B, cent.shape[1], _KP, g.shape[2])
    p1 = jnp.stack([layers[0]['g'], layers[0]['be']])
    p2 = jnp.stack([layers[1]['g'], layers[1]['be']])
    p3 = jnp.stack([layers[2]['g'], layers[2]['be']])
    return _mlp_call(gn, c1m, nd2, layers[1]['W'], layers[1]['b'][None, :],
                     layers[2]['W'], layers[2]['b'][None, :], p1, p2, p3,
                     r, mblk)


def kernel(positions, params):
    pos_t = jnp.transpose(positions, (2, 0, 1))
    s1, s2, s3, s4 = _fps_call(pos_t)
    c1 = jnp.transpose(s1, (1, 2, 0))
    c2 = jnp.transpose(s2, (1, 2, 0))
    c3 = jnp.transpose(s3, (1, 2, 0))
    c4 = jnp.transpose(s4, (1, 2, 0))
    pt0 = jnp.transpose(positions, (0, 2, 1))
    pt1 = jnp.transpose(s1, (1, 0, 2))
    pt2 = jnp.transpose(s2, (1, 0, 2))
    pt3 = jnp.transpose(s3, (1, 0, 2))

    f1 = _sa_level(positions, c1, pt0, 3.0, params['sa1'], 208, False)
    f2 = _sa_level(jnp.concatenate([f1, c1], axis=2), c2, pt1, 4.0,
                   params['sa2'], 205, True)
    f3 = _sa_level(jnp.concatenate([f2, c2], axis=2), c3, pt2, 5.0,
                   params['sa3'], 52, True)
    f4 = _sa_level(jnp.concatenate([f3, c3], axis=2), c4, pt3, 7.0,
                   params['sa4'], 13, True)

    return _head_call(f4, c4, params['last'], params['linW'], params['linb'])
